# baseline (speedup 1.0000x reference)
import functools

import jax
import jax.numpy as jnp
from jax import lax
from jax.experimental import pallas as pl
from jax.experimental.pallas import tpu as pltpu
from jax.experimental.pallas import tpu_sc as plsc

N_ITEMS = 1000000
EMBED_DIM = 64
BATCH = 16384

NC = 2
NS = 16
NW = NC * NS
B_PER_W = BATCH // NW
CHUNK = 128
N_CHUNKS = B_PER_W // CHUNK


def _sc_gather(ids_2d, table):
    mesh = plsc.VectorSubcoreMesh(core_axis_name="c", subcore_axis_name="s")

    @functools.partial(
        pl.kernel,
        mesh=mesh,
        compiler_params=pltpu.CompilerParams(use_tc_tiling_on_sc=False),
        out_type=jax.ShapeDtypeStruct((BATCH, EMBED_DIM), jnp.float32),
        scratch_types=[
            pltpu.VMEM((N_CHUNKS, CHUNK), jnp.int32),
            pltpu.VMEM((B_PER_W, EMBED_DIM), jnp.float32),
            pltpu.SemaphoreType.DMA,
        ],
    )
    def gather_k(ids_hbm, table_hbm, out_hbm, idx_v, rows_v, sem):
        wid = lax.axis_index("s") * NC + lax.axis_index("c")
        pltpu.sync_copy(ids_hbm.at[pl.ds(wid * N_CHUNKS, N_CHUNKS)], idx_v)
        copies = [
            pltpu.async_copy(
                table_hbm.at[idx_v.at[j]],
                rows_v.at[pl.ds(j * CHUNK, CHUNK)],
                sem,
            )
            for j in range(N_CHUNKS)
        ]
        for cp in copies:
            cp.wait()
        pltpu.sync_copy(rows_v, out_hbm.at[pl.ds(wid * B_PER_W, B_PER_W)])

    return gather_k(ids_2d, table)


def _tc_body(emb_ref, feat_ref, w1t_ref, b1_ref, w2t_ref, b2_ref,
             we_ref, wf_ref, bc_ref, out_ref):
    f = feat_ref[...]
    h = jnp.maximum(
        jnp.dot(f, w1t_ref[...], preferred_element_type=jnp.float32)
        + b1_ref[...], 0.0)
    f2 = (jnp.dot(h, w2t_ref[...], preferred_element_type=jnp.float32)
          + b2_ref[...])
    o = (jnp.dot(emb_ref[...], we_ref[...], preferred_element_type=jnp.float32)
         + jnp.dot(f2, wf_ref[...], preferred_element_type=jnp.float32)
         + bc_ref[...])
    s = jnp.sum(o * o, axis=1, keepdims=True)
    out_ref[...] = o * lax.rsqrt(jnp.maximum(s, 1e-24))


def _tc_dense(emb, feats, w1t, b1r, w2t, b2r, we, wf, bcr):
    tb = 2048
    grid = BATCH // tb
    full = lambda shape: pl.BlockSpec(shape, lambda i: (0, 0))
    return pl.pallas_call(
        _tc_body,
        grid=(grid,),
        in_specs=[
            pl.BlockSpec((tb, EMBED_DIM), lambda i: (i, 0)),
            pl.BlockSpec((tb, 4), lambda i: (i, 0)),
            full((4, 32)),
            full((1, 32)),
            full((32, EMBED_DIM)),
            full((1, EMBED_DIM)),
            full((EMBED_DIM, EMBED_DIM)),
            full((EMBED_DIM, EMBED_DIM)),
            full((1, EMBED_DIM)),
        ],
        out_specs=pl.BlockSpec((tb, EMBED_DIM), lambda i: (i, 0)),
        out_shape=jax.ShapeDtypeStruct((BATCH, EMBED_DIM), jnp.float32),
    )(emb, feats, w1t, b1r, w2t, b2r, we, wf, bcr)


def kernel(item_ids, item_features, emb_table, W1, b1, W2, b2, Wc, bc):
    ids_2d = item_ids.astype(jnp.int32).reshape(NW * N_CHUNKS, CHUNK)
    emb = _sc_gather(ids_2d, emb_table)
    return _tc_dense(
        emb,
        item_features,
        W1.T,
        b1.reshape(1, 32),
        W2.T,
        b2.reshape(1, EMBED_DIM),
        Wc[:, :EMBED_DIM].T,
        Wc[:, EMBED_DIM:].T,
        bc.reshape(1, EMBED_DIM),
    )


# speedup vs baseline: 1.6679x; 1.6679x over previous
import functools

import jax
import jax.numpy as jnp
from jax import lax
from jax.experimental import pallas as pl
from jax.experimental.pallas import tpu as pltpu
from jax.experimental.pallas import tpu_sc as plsc

N_ITEMS = 1000000
EMBED_DIM = 64
BATCH = 16384

NC = 2
NS = 16
NW = NC * NS
B_PER_W = BATCH // NW
GRP = 16
N_GROUPS = N_ITEMS // GRP
W = 32
NWIN = B_PER_W // W
LANES = 16


def _sc_gather(ids_hbm_arr, table):
    mesh = plsc.VectorSubcoreMesh(core_axis_name="c", subcore_axis_name="s")

    @functools.partial(
        pl.kernel,
        mesh=mesh,
        out_type=jax.ShapeDtypeStruct((BATCH, EMBED_DIM), jnp.float32),
        scratch_types=[
            pltpu.VMEM((B_PER_W,), jnp.int32),
            pltpu.VMEM((B_PER_W, EMBED_DIM), jnp.float32),
            pltpu.SemaphoreType.DMA,
        ],
    )
    def gather_k(ids_hbm, table_hbm, out_hbm, ids_v, rows_v, sem):
        wid = lax.axis_index("s") * NC + lax.axis_index("c")
        base = wid * B_PER_W
        pltpu.sync_copy(ids_hbm.at[pl.ds(base, B_PER_W)], ids_v)

        n_blk = B_PER_W // LANES

        def fire(blk):
            idv = ids_v[pl.ds(blk * LANES, LANES)]
            for l in range(LANES):
                row = idv[l]
                pltpu.async_copy(
                    table_hbm.at[pl.ds(row, 1)],
                    rows_v.at[pl.ds(blk * LANES + l, 1)],
                    sem,
                )

        def drain(blk):
            pltpu.make_async_copy(
                table_hbm.at[pl.ds(0, LANES)],
                rows_v.at[pl.ds(blk * LANES, LANES)],
                sem,
            ).wait()

        fire(0)

        def body(i, _):
            @pl.when(i + 1 < n_blk)
            def _():
                fire(i + 1)
            drain(i)
            return 0

        lax.fori_loop(0, n_blk, body, 0, unroll=False)
        pltpu.sync_copy(rows_v, out_hbm.at[pl.ds(base, B_PER_W)])

    return gather_k(ids_hbm_arr, table)


def _tc_body(emb_ref, feat_ref, w1t_ref, b1_ref, w2t_ref, b2_ref,
             we_ref, wf_ref, bc_ref, out_ref):
    f = feat_ref[...]
    h = jnp.maximum(
        jnp.dot(f, w1t_ref[...], preferred_element_type=jnp.float32)
        + b1_ref[...], 0.0)
    f2 = (jnp.dot(h, w2t_ref[...], preferred_element_type=jnp.float32)
          + b2_ref[...])
    o = (jnp.dot(emb_ref[...], we_ref[...], preferred_element_type=jnp.float32)
         + jnp.dot(f2, wf_ref[...], preferred_element_type=jnp.float32)
         + bc_ref[...])
    s = jnp.sum(o * o, axis=1, keepdims=True)
    out_ref[...] = o * lax.rsqrt(jnp.maximum(s, 1e-24))


def _tc_dense(emb, feats, w1t, b1r, w2t, b2r, we, wf, bcr):
    tb = 2048
    grid = BATCH // tb
    full = lambda shape: pl.BlockSpec(shape, lambda i: (0, 0))
    return pl.pallas_call(
        _tc_body,
        grid=(grid,),
        in_specs=[
            pl.BlockSpec((tb, EMBED_DIM), lambda i: (i, 0)),
            pl.BlockSpec((tb, 4), lambda i: (i, 0)),
            full((4, 32)),
            full((1, 32)),
            full((32, EMBED_DIM)),
            full((1, EMBED_DIM)),
            full((EMBED_DIM, EMBED_DIM)),
            full((EMBED_DIM, EMBED_DIM)),
            full((1, EMBED_DIM)),
        ],
        out_specs=pl.BlockSpec((tb, EMBED_DIM), lambda i: (i, 0)),
        out_shape=jax.ShapeDtypeStruct((BATCH, EMBED_DIM), jnp.float32),
    )(emb, feats, w1t, b1r, w2t, b2r, we, wf, bcr)


def kernel(item_ids, item_features, emb_table, W1, b1, W2, b2, Wc, bc):
    emb = _sc_gather(item_ids.astype(jnp.int32), emb_table)
    return _tc_dense(
        emb,
        item_features,
        W1.T,
        b1.reshape(1, 32),
        W2.T,
        b2.reshape(1, EMBED_DIM),
        Wc[:, :EMBED_DIM].T,
        Wc[:, EMBED_DIM:].T,
        bc.reshape(1, EMBED_DIM),
    )
